# baseline (speedup 1.0000x reference)
import functools

import jax
import jax.numpy as jnp
import numpy as np
from jax import lax
from jax.experimental import pallas as pl
from jax.experimental.pallas import tpu as pltpu
from jax.experimental.pallas import tpu_sc as plsc

_K = 32
_NBUF = 3


def _sc_body(h2_hbm, gidx_hbm, pos_hbm, emb_hbm, out_hbm,
             buf, gidx_v, posidx_v, embrow_v, in_sems, out_sems, aux_sem):
    bsz = out_hbm.shape[0]
    s = out_hbm.shape[1] - 1
    wpb = 8
    rows_pw = s // wpb
    nchunk = rows_pw // _K
    wid = lax.axis_index("s") * 2 + lax.axis_index("c")
    b = wid // wpb
    j = wid % wpb
    base = j * rows_pw

    pltpu.sync_copy(gidx_hbm.at[wid], gidx_v)
    pltpu.sync_copy(pos_hbm, posidx_v)
    emb_cp = pltpu.make_async_copy(emb_hbm.at[posidx_v], embrow_v, aux_sem)
    emb_cp.start()

    def in_cp(c):
        return pltpu.make_async_copy(
            h2_hbm.at[gidx_v.at[c]], buf.at[c % _NBUF], in_sems.at[c % _NBUF])

    def out_cp(c):
        dst = out_hbm.at[b, pl.ds(base + c * _K, _K), :]
        return pltpu.make_async_copy(buf.at[c % _NBUF], dst, out_sems.at[c % _NBUF])

    for c in range(min(_NBUF, nchunk)):
        in_cp(c).start()
    for c in range(nchunk):
        in_cp(c).wait()
        out_cp(c).start()
        nxt = c + _NBUF
        if nxt < nchunk:
            out_cp(nxt - _NBUF).wait()
            in_cp(nxt).start()
    for c in range(max(0, nchunk - _NBUF + 1), nchunk):
        out_cp(c).wait()
    emb_cp.wait()

    @pl.when(j == 0)
    def _():
        pltpu.sync_copy(embrow_v.at[pl.ds(0, 1)], out_hbm.at[b, pl.ds(0, 1), :])

    @pl.when(j == wpb - 1)
    def _():
        g = pltpu.make_async_copy(
            h2_hbm.at[gidx_v.at[nchunk]], buf.at[0], in_sems.at[0])
        g.start()
        g.wait()
        pltpu.sync_copy(buf.at[0, pl.ds(0, 1)], out_hbm.at[b, pl.ds(s, 1), :])


def _gather_indices(b, s):
    w = np.arange(32)
    bi, j = w // 8, w % 8
    c = np.arange(9)
    i = np.arange(_K)
    r = (j[:, None, None] * (s // 8) + c[None, :, None] * _K
         + i[None, None, :] - 1)
    idx = bi[:, None, None] * s + np.clip(r, 0, s - 1)
    idx[:, 8, :] = bi[:, None] * s + (s - 1)
    return jnp.asarray(idx, dtype=jnp.int32)


def kernel(hidden_states, position, embeddings):
    b, s, d = hidden_states.shape
    h2 = hidden_states.reshape(b * s, d)
    gidx = _gather_indices(b, s)
    pos_arr = jnp.broadcast_to(jnp.asarray(position, jnp.int32), (16,))
    mesh = plsc.VectorSubcoreMesh(core_axis_name="c", subcore_axis_name="s")
    run = functools.partial(
        pl.kernel,
        out_type=jax.ShapeDtypeStruct((b, s + 1, d), hidden_states.dtype),
        mesh=mesh,
        scratch_types=[
            pltpu.VMEM((_NBUF, _K, d), hidden_states.dtype),
            pltpu.VMEM((9, _K), jnp.int32),
            pltpu.VMEM((16,), jnp.int32),
            pltpu.VMEM((16, d), hidden_states.dtype),
            pltpu.SemaphoreType.DMA((_NBUF,)),
            pltpu.SemaphoreType.DMA((_NBUF,)),
            pltpu.SemaphoreType.DMA,
        ],
    )(_sc_body)
    return run(h2, gidx, pos_arr, embeddings)


# speedup vs baseline: 1.6743x; 1.6743x over previous
import jax
import jax.numpy as jnp
from jax.experimental import pallas as pl
from jax.experimental.pallas import tpu as pltpu


def _concat_body(pos_ref, h_ref, emb_ref, out_ref):
    s = h_ref.shape[1]
    out_ref[0, pl.ds(1, s), :] = h_ref[0]
    out_ref[0, pl.ds(0, 1), :] = emb_ref[pl.ds(pos_ref[0], 1), :]


def kernel(hidden_states, position, embeddings):
    b, s, d = hidden_states.shape
    depth = embeddings.shape[0]
    pos_arr = jnp.asarray(position, jnp.int32).reshape((1,))
    return pl.pallas_call(
        _concat_body,
        grid=(b,),
        out_shape=jax.ShapeDtypeStruct((b, s + 1, d), hidden_states.dtype),
        in_specs=[
            pl.BlockSpec(memory_space=pltpu.SMEM),
            pl.BlockSpec((1, s, d), lambda i: (i, 0, 0)),
            pl.BlockSpec((depth, d), lambda i: (0, 0)),
        ],
        out_specs=pl.BlockSpec((1, s + 1, d), lambda i: (i, 0, 0)),
    )(pos_arr, hidden_states, embeddings)
